# baseline (speedup 1.0000x reference)
import functools

import jax
import jax.numpy as jnp
from jax import lax
from jax.experimental import pallas as pl
from jax.experimental.pallas import tpu as pltpu
from jax.experimental.pallas import tpu_sc as plsc

N, D = 100000, 128
BS = 20000
G = 16


def _nw_sc_body(w_hbm, nw_hbm, rows_v):
    cid = lax.axis_index("c")
    sid = lax.axis_index("s")
    wid = sid * 2 + cid

    @pl.when(wid < 4)
    def _():
        src = pl.multiple_of(wid * 1000, 8)
        dst = pl.multiple_of(wid * G, 8)
        pltpu.sync_copy(w_hbm.at[pl.ds(src, G)], rows_v)
        eps = (wid.astype(jnp.float32) + 1.0) * 0.5
        rows = lax.iota(jnp.int32, G)
        def _accum(c, ss):
            col = jnp.full((G,), c, jnp.int32)
            v = plsc.load_gather(rows_v, [rows, col])
            return ss + v * v
        ss = lax.fori_loop(0, D, _accum, jnp.zeros((G,), jnp.float32))
        i = plsc.bitcast(ss, jnp.int32)
        i = jnp.int32(0x5F3759DF) - (i >> 1)
        r = plsc.bitcast(i, jnp.float32)
        for _ in range(3):
            r = r * (1.5 - 0.5 * ss * r * r)
        scale = jnp.where(ss <= eps * eps, jnp.float32(1.0), eps * r)

        def _rescale(c, carry):
            col = jnp.full((G,), c, jnp.int32)
            v = plsc.load_gather(rows_v, [rows, col])
            plsc.store_scatter(rows_v, [rows, col], v * scale)
            return carry
        lax.fori_loop(0, D, _rescale, jnp.int32(0))
        pltpu.sync_copy(rows_v, nw_hbm.at[pl.ds(dst, G)])


def _nw_sc(W):
    sc_kernel = functools.partial(
        pl.kernel,
        out_type=jax.ShapeDtypeStruct((4 * G, D), jnp.float32),
        mesh=plsc.VectorSubcoreMesh(core_axis_name="c", subcore_axis_name="s"),
        scratch_types=[pltpu.VMEM((G, D), jnp.float32)],
        compiler_params=pltpu.CompilerParams(needs_layout_passes=False),
    )(_nw_sc_body)
    return sc_kernel(W)


def _fused_body(x_ref, w_ref, nw_ref, o_ref, wout_ref):
    pid = pl.program_id(0)
    w = w_ref[...]
    wout_ref[...] = w
    o_ref[...] = x_ref[...] + w

    @pl.when(pid == 0)
    def _():
        for g in range(4):
            off = g * 1000
            nwg = nw_ref[g * G:(g + 1) * G, :]
            wout_ref[off:off + G, :] = nwg
            o_ref[off:off + G, :] = x_ref[off:off + G, :] + nwg


def _fused_tc(x, W, nw):
    return pl.pallas_call(
        _fused_body,
        grid=(N // BS,),
        in_specs=[
            pl.BlockSpec((BS, D), lambda i: (i, 0)),
            pl.BlockSpec((BS, D), lambda i: (i, 0)),
            pl.BlockSpec((4 * G, D), lambda i: (0, 0)),
        ],
        out_specs=[
            pl.BlockSpec((BS, D), lambda i: (i, 0)),
            pl.BlockSpec((BS, D), lambda i: (i, 0)),
        ],
        out_shape=[
            jax.ShapeDtypeStruct((N, D), jnp.float32),
            jax.ShapeDtypeStruct((N, D), jnp.float32),
        ],
        compiler_params=pltpu.CompilerParams(
            dimension_semantics=("parallel",),
            vmem_limit_bytes=110_000_000,
        ),
    )(x, W, nw)


def kernel(x, W):
    nw = _nw_sc(W)
    out, Wout = _fused_tc(x, W, nw)
    return (out, Wout)


# speedup vs baseline: 1.0180x; 1.0180x over previous
import functools

import jax
import jax.numpy as jnp
from jax import lax
from jax.experimental import pallas as pl
from jax.experimental.pallas import tpu as pltpu
from jax.experimental.pallas import tpu_sc as plsc

N, D = 100000, 128
BS = 20000
G = 16


def _nw_sc_body(w_hbm, nw_hbm, rows_v):
    cid = lax.axis_index("c")
    sid = lax.axis_index("s")
    wid = sid * 2 + cid

    @pl.when(wid < 4)
    def _():
        src = pl.multiple_of(wid * 1000, 8)
        dst = pl.multiple_of(wid * G, 8)
        pltpu.sync_copy(w_hbm.at[pl.ds(src, G)], rows_v)
        eps = (wid.astype(jnp.float32) + 1.0) * 0.5
        rows = lax.iota(jnp.int32, G)
        @plsc.parallel_loop(0, D, unroll=8, carry=jnp.zeros((G,), jnp.float32))
        def ss(c, acc):
            col = jnp.full((G,), c, jnp.int32)
            v = plsc.load_gather(rows_v, [rows, col])
            return acc + v * v
        i = plsc.bitcast(ss, jnp.int32)
        i = jnp.int32(0x5F3759DF) - (i >> 1)
        r = plsc.bitcast(i, jnp.float32)
        for _ in range(3):
            r = r * (1.5 - 0.5 * ss * r * r)
        scale = jnp.where(ss <= eps * eps, jnp.float32(1.0), eps * r)

        @plsc.parallel_loop(0, D, unroll=8)
        def _rescale(c):
            col = jnp.full((G,), c, jnp.int32)
            v = plsc.load_gather(rows_v, [rows, col])
            plsc.store_scatter(rows_v, [rows, col], v * scale)
        pltpu.sync_copy(rows_v, nw_hbm.at[pl.ds(dst, G)])


def _nw_sc(W):
    sc_kernel = functools.partial(
        pl.kernel,
        out_type=jax.ShapeDtypeStruct((4 * G, D), jnp.float32),
        mesh=plsc.VectorSubcoreMesh(core_axis_name="c", subcore_axis_name="s"),
        scratch_types=[pltpu.VMEM((G, D), jnp.float32)],
        compiler_params=pltpu.CompilerParams(needs_layout_passes=False),
    )(_nw_sc_body)
    return sc_kernel(W)


def _fused_body(x_ref, w_ref, nw_ref, o_ref, wout_ref):
    pid = pl.program_id(0)
    w = w_ref[...]
    wout_ref[...] = w
    o_ref[...] = x_ref[...] + w

    @pl.when(pid == 0)
    def _():
        for g in range(4):
            off = g * 1000
            nwg = nw_ref[g * G:(g + 1) * G, :]
            wout_ref[off:off + G, :] = nwg
            o_ref[off:off + G, :] = x_ref[off:off + G, :] + nwg


def _fused_tc(x, W, nw):
    return pl.pallas_call(
        _fused_body,
        grid=(N // BS,),
        in_specs=[
            pl.BlockSpec((BS, D), lambda i: (i, 0)),
            pl.BlockSpec((BS, D), lambda i: (i, 0)),
            pl.BlockSpec((4 * G, D), lambda i: (0, 0)),
        ],
        out_specs=[
            pl.BlockSpec((BS, D), lambda i: (i, 0)),
            pl.BlockSpec((BS, D), lambda i: (i, 0)),
        ],
        out_shape=[
            jax.ShapeDtypeStruct((N, D), jnp.float32),
            jax.ShapeDtypeStruct((N, D), jnp.float32),
        ],
        compiler_params=pltpu.CompilerParams(
            dimension_semantics=("parallel",),
            vmem_limit_bytes=110_000_000,
        ),
    )(x, W, nw)


def kernel(x, W):
    nw = _nw_sc(W)
    out, Wout = _fused_tc(x, W, nw)
    return (out, Wout)


# speedup vs baseline: 1.0353x; 1.0170x over previous
import functools

import jax
import jax.numpy as jnp
from jax import lax
from jax.experimental import pallas as pl
from jax.experimental.pallas import tpu as pltpu
from jax.experimental.pallas import tpu_sc as plsc

N, D = 100000, 128
BS = 10000
G = 16


def _nw_sc_body(w_hbm, nw_hbm, rows_v):
    wid = lax.axis_index("s")

    @pl.when(wid < 4)
    def _():
        src = pl.multiple_of(wid * 1000, 8)
        dst = pl.multiple_of(wid * G, 8)
        pltpu.sync_copy(w_hbm.at[pl.ds(src, G)], rows_v)
        eps = (wid.astype(jnp.float32) + 1.0) * 0.5
        rows = lax.iota(jnp.int32, G)
        @plsc.parallel_loop(0, D, unroll=8, carry=jnp.zeros((G,), jnp.float32))
        def ss(c, acc):
            col = jnp.full((G,), c, jnp.int32)
            v = plsc.load_gather(rows_v, [rows, col])
            return acc + v * v
        i = plsc.bitcast(ss, jnp.int32)
        i = jnp.int32(0x5F3759DF) - (i >> 1)
        r = plsc.bitcast(i, jnp.float32)
        for _ in range(3):
            r = r * (1.5 - 0.5 * ss * r * r)
        scale = jnp.where(ss <= eps * eps, jnp.float32(1.0), eps * r)

        @plsc.parallel_loop(0, D, unroll=8)
        def _rescale(c):
            col = jnp.full((G,), c, jnp.int32)
            v = plsc.load_gather(rows_v, [rows, col])
            plsc.store_scatter(rows_v, [rows, col], v * scale)
        pltpu.sync_copy(rows_v, nw_hbm.at[pl.ds(dst, G)])


def _nw_sc(W):
    sc_kernel = functools.partial(
        pl.kernel,
        out_type=jax.ShapeDtypeStruct((4 * G, D), jnp.float32),
        mesh=plsc.VectorSubcoreMesh(core_axis_name="c", subcore_axis_name="s",
                                    num_cores=1),
        scratch_types=[pltpu.VMEM((G, D), jnp.float32)],
        compiler_params=pltpu.CompilerParams(needs_layout_passes=False),
    )(_nw_sc_body)
    return sc_kernel(W)


def _fused_body(x_ref, w_ref, nw_ref, o_ref, wout_ref):
    pid = pl.program_id(0)
    w = w_ref[...]
    wout_ref[...] = w
    o_ref[...] = x_ref[...] + w

    @pl.when(pid == 0)
    def _():
        for g in range(4):
            off = g * 1000
            nwg = nw_ref[g * G:(g + 1) * G, :]
            wout_ref[off:off + G, :] = nwg
            o_ref[off:off + G, :] = x_ref[off:off + G, :] + nwg


def _fused_tc(x, W, nw):
    return pl.pallas_call(
        _fused_body,
        grid=(N // BS,),
        in_specs=[
            pl.BlockSpec((BS, D), lambda i: (i, 0)),
            pl.BlockSpec((BS, D), lambda i: (i, 0)),
            pl.BlockSpec((4 * G, D), lambda i: (0, 0)),
        ],
        out_specs=[
            pl.BlockSpec((BS, D), lambda i: (i, 0)),
            pl.BlockSpec((BS, D), lambda i: (i, 0)),
        ],
        out_shape=[
            jax.ShapeDtypeStruct((N, D), jnp.float32),
            jax.ShapeDtypeStruct((N, D), jnp.float32),
        ],
        compiler_params=pltpu.CompilerParams(
            dimension_semantics=("parallel",),
        ),
    )(x, W, nw)


def kernel(x, W):
    nw = _nw_sc(W)
    out, Wout = _fused_tc(x, W, nw)
    return (out, Wout)
